# baseline (speedup 1.0000x reference)
import functools

import jax
import jax.numpy as jnp
from jax import lax
from jax.experimental import pallas as pl
from jax.experimental.pallas import tpu as pltpu
from jax.experimental.pallas import tpu_sc as plsc

VOCAB = 8192
DIM = 8192
BATCH = 16384

NUM_CORES = 2
NUM_SUBCORES = 16
NW = NUM_CORES * NUM_SUBCORES
BPW = BATCH // NW
CHUNK = 2
NBUF = 4
NCHUNKS = BPW // CHUNK
NT = NCHUNKS // NBUF


def _sc_gather(x, table):
    mesh = plsc.VectorSubcoreMesh(core_axis_name="c", subcore_axis_name="s")

    @functools.partial(
        pl.kernel,
        mesh=mesh,
        out_type=jax.ShapeDtypeStruct((BATCH, DIM), jnp.float32),
        scratch_types=[
            pltpu.VMEM((NCHUNKS, CHUNK), jnp.int32),
            pltpu.VMEM((NBUF, CHUNK, DIM), jnp.float32),
        ] + [pltpu.SemaphoreType.DMA] * NBUF,
    )
    def k(x_hbm, table_hbm, out_hbm, idx_v, bufs, *gsems):
        wid = lax.axis_index("s") * NUM_CORES + lax.axis_index("c")
        base = wid * BPW
        pltpu.sync_copy(x_hbm.at[wid], idx_v)

        def start_gather(g, b):
            pltpu.async_copy(
                table_hbm.at[idx_v.at[g]],
                bufs.at[b],
                gsems[b],
            )

        for b in range(NBUF):
            start_gather(b, b)

        def body(t, carry):
            for b in range(NBUF):
                g = t * NBUF + b
                pltpu.make_async_copy(
                    table_hbm.at[idx_v.at[0]],
                    bufs.at[b],
                    gsems[b],
                ).wait()
                pltpu.sync_copy(
                    bufs.at[b],
                    out_hbm.at[pl.ds(base + g * CHUNK, CHUNK)],
                )

                @pl.when(g + NBUF < NCHUNKS)
                def _():
                    start_gather(g + NBUF, b)

            return carry

        lax.fori_loop(0, NT, body, 0)

    return k(x, table)


def kernel(x, table):
    x3 = x.astype(jnp.int32).reshape(NW, NCHUNKS, CHUNK)
    return _sc_gather(x3, table)


# speedup vs baseline: 1.0042x; 1.0042x over previous
import functools

import jax
import jax.numpy as jnp
from jax import lax
from jax.experimental import pallas as pl
from jax.experimental.pallas import tpu as pltpu
from jax.experimental.pallas import tpu_sc as plsc

VOCAB = 8192
DIM = 8192
BATCH = 16384

NUM_CORES = 2
NUM_SUBCORES = 16
NW = NUM_CORES * NUM_SUBCORES
BPW = BATCH // NW
CHUNK = 4
NBUF = 2
NCHUNKS = BPW // CHUNK
NT = NCHUNKS // NBUF


def _sc_gather(x, table):
    mesh = plsc.VectorSubcoreMesh(core_axis_name="c", subcore_axis_name="s")

    @functools.partial(
        pl.kernel,
        mesh=mesh,
        out_type=jax.ShapeDtypeStruct((BATCH, DIM), jnp.float32),
        scratch_types=[
            pltpu.VMEM((NCHUNKS, CHUNK), jnp.int32),
            pltpu.VMEM((NBUF, CHUNK, DIM), jnp.float32),
        ] + [pltpu.SemaphoreType.DMA] * NBUF,
    )
    def k(x_hbm, table_hbm, out_hbm, idx_v, bufs, *gsems):
        wid = lax.axis_index("s") * NUM_CORES + lax.axis_index("c")
        base = wid * BPW
        pltpu.sync_copy(x_hbm.at[wid], idx_v)

        def start_gather(g, b):
            pltpu.async_copy(
                table_hbm.at[idx_v.at[g]],
                bufs.at[b],
                gsems[b],
            )

        for b in range(NBUF):
            start_gather(b, b)

        def body(t, carry):
            for b in range(NBUF):
                g = t * NBUF + b
                pltpu.make_async_copy(
                    table_hbm.at[idx_v.at[0]],
                    bufs.at[b],
                    gsems[b],
                ).wait()
                pltpu.sync_copy(
                    bufs.at[b],
                    out_hbm.at[pl.ds(base + g * CHUNK, CHUNK)],
                )

                @pl.when(g + NBUF < NCHUNKS)
                def _():
                    start_gather(g + NBUF, b)

            return carry

        lax.fori_loop(0, NT, body, 0)

    return k(x, table)


def kernel(x, table):
    x3 = x.astype(jnp.int32).reshape(NW, NCHUNKS, CHUNK)
    return _sc_gather(x3, table)
